# baseline (speedup 1.0000x reference)
import functools

import jax
import jax.numpy as jnp
from jax import lax
from jax.experimental import pallas as pl
from jax.experimental.pallas import tpu as pltpu
from jax.experimental.pallas import tpu_sc as plsc

NC = 2
NS = 16
LANES = 16
NW = NC * NS

B = 4096
L = 200
D = 128
DW = D // 2
PARTS = (96, 104)
B_PER_W = B // NW
D_VECS = DW // LANES

HI_MASK = -65536

VOCAB = 100000
ROWS_W = VOCAB // NW
CH = 125
N_CH = ROWS_W // CH


def _rne_bf16(t):
    sixteen = jnp.uint32(16)
    return t + jnp.uint32(32767) + (lax.shift_right_logical(t, sixteen)
                                    & jnp.uint32(1))


def _pack_kernel(table_hbm, packed_hbm,
                 in_a, in_b, out_a, out_b,
                 semi_a, semi_b, semo_a, semo_b):
    wid = lax.axis_index("s") * NC + lax.axis_index("c")
    base = wid * ROWS_W

    def start_in(c, buf, sem):
        pltpu.make_async_copy(
            table_hbm.at[pl.ds(base + c * CH, CH)], buf, sem).start()

    def wait_in(buf, sem):
        pltpu.make_async_copy(
            table_hbm.at[pl.ds(base, CH)], buf, sem).wait()

    def start_out(c, buf, sem):
        pltpu.make_async_copy(
            buf, packed_hbm.at[pl.ds(base + c * CH, CH)], sem).start()

    def wait_out(buf, sem):
        pltpu.make_async_copy(
            buf, packed_hbm.at[pl.ds(base, CH)], sem).wait()

    def compute(src, dst):
        def body(r, _):
            for c in range(D_VECS):
                t0 = plsc.bitcast(src[r, pl.ds(c * LANES, LANES)], jnp.uint32)
                t1 = plsc.bitcast(src[r, pl.ds(DW + c * LANES, LANES)],
                                  jnp.uint32)
                rlo = lax.shift_right_logical(t0, jnp.uint32(16))
                rhi = t1 & jnp.uint32(4294901760)
                dst[r, pl.ds(c * LANES, LANES)] = plsc.bitcast(rlo | rhi,
                                                               jnp.int32)
            return 0
        lax.fori_loop(0, CH, body, 0, unroll=5)

    start_in(0, in_a, semi_a)

    def pair_body(i, _):
        c0 = 2 * i
        c1 = 2 * i + 1

        @pl.when(c1 < N_CH)
        def _():
            start_in(c1, in_b, semi_b)
        wait_in(in_a, semi_a)

        @pl.when(c0 >= 2)
        def _():
            wait_out(out_a, semo_a)
        compute(in_a, out_a)
        start_out(c0, out_a, semo_a)

        @pl.when(c0 + 2 < N_CH)
        def _():
            start_in(c0 + 2, in_a, semi_a)

        @pl.when(c1 < N_CH)
        def _():
            wait_in(in_b, semi_b)

            @pl.when(c1 >= 2)
            def _():
                wait_out(out_b, semo_b)
            compute(in_b, out_b)
            start_out(c1, out_b, semo_b)
        return 0

    lax.fori_loop(0, (N_CH + 1) // 2, pair_body, 0)
    wait_out(out_a, semo_a)
    wait_out(out_b, semo_b)


def _cbow_kernel(words_hbm, bs_hbm, table_hbm, out_hbm,
                 idx_v, bs_v, recip_v, rows_a, rows_b, rows_c, rows_d, out_v,
                 sem_a, sem_b, sem_c, sem_d):
    wid = lax.axis_index("s") * NC + lax.axis_index("c")
    base = wid * B_PER_W

    pltpu.sync_copy(words_hbm.at[pl.ds(base * L, B_PER_W * L)], idx_v)
    pltpu.sync_copy(bs_hbm.at[pl.ds(base, B_PER_W)], bs_v)

    for c in range(B_PER_W // LANES):
        bsf = bs_v[pl.ds(c * LANES, LANES)].astype(jnp.float32)
        recip_v[pl.ds(c * LANES, LANES)] = 1.0 / bsf

    def start_gather(s, rows_ref, sem):
        off = 0
        for n in PARTS:
            pltpu.make_async_copy(
                table_hbm.at[idx_v.at[pl.ds(s * L + off, n)]],
                rows_ref.at[pl.ds(off, n)],
                sem,
            ).start()
            off += n

    def wait_gather(rows_ref, sem):
        off = 0
        for n in PARTS:
            pltpu.make_async_copy(
                table_hbm.at[idx_v.at[pl.ds(off, n)]],
                rows_ref.at[pl.ds(off, n)],
                sem,
            ).wait()
            off += n

    def accum_and_store(s, rows_ref):
        def body(p, accs):
            new = list(accs)
            for c in range(D_VECS):
                wa = plsc.bitcast(rows_ref[2 * p, pl.ds(c * LANES, LANES)],
                                  jnp.bfloat16)
                wb = plsc.bitcast(rows_ref[2 * p + 1, pl.ds(c * LANES, LANES)],
                                  jnp.bfloat16)
                t = plsc.bitcast(wa + wb, jnp.int32)
                lo = plsc.bitcast(lax.shift_left(t, 16), jnp.float32)
                hi = plsc.bitcast(t & HI_MASK, jnp.float32)
                new[2 * c] = new[2 * c] + lo
                new[2 * c + 1] = new[2 * c + 1] + hi
            return tuple(new)

        accs = lax.fori_loop(
            0, L // 2, body,
            tuple(jnp.zeros((LANES,), jnp.float32) for _ in range(2 * D_VECS)),
            unroll=2)
        r = plsc.load_gather(recip_v, [jnp.full((LANES,), s, dtype=jnp.int32)])
        for c in range(D_VECS):
            out_v[s, pl.ds(c * LANES, LANES)] = accs[2 * c] * r
            out_v[s, pl.ds(DW + c * LANES, LANES)] = accs[2 * c + 1] * r

    start_gather(0, rows_a, sem_a)
    start_gather(1, rows_b, sem_b)
    start_gather(2, rows_c, sem_c)

    bufs = ((rows_a, sem_a), (rows_b, sem_b), (rows_c, sem_c),
            (rows_d, sem_d))

    def quad_body(i, _):
        s = 4 * i
        start_gather(s + 3, rows_d, sem_d)
        for k, (buf, sem) in enumerate(bufs):
            wait_gather(buf, sem)
            accum_and_store(s + k, buf)

            if k < 3:
                @pl.when(s + k + 4 < B_PER_W)
                def _():
                    start_gather(s + k + 4, buf, sem)
        return 0

    lax.fori_loop(0, B_PER_W // 4, quad_body, 0)

    pltpu.sync_copy(out_v, out_hbm.at[pl.ds(base, B_PER_W)])


@jax.jit
def _cbow(words, bs, table):
    pack_run = pl.kernel(
        _pack_kernel,
        out_type=jax.ShapeDtypeStruct((VOCAB, DW), jnp.int32),
        mesh=plsc.VectorSubcoreMesh(core_axis_name="c", subcore_axis_name="s"),
        compiler_params=pltpu.CompilerParams(needs_layout_passes=False,
                                             use_tc_tiling_on_sc=False),
        scratch_types=[
            pltpu.VMEM((CH, D), jnp.float32),
            pltpu.VMEM((CH, D), jnp.float32),
            pltpu.VMEM((CH, DW), jnp.int32),
            pltpu.VMEM((CH, DW), jnp.int32),
            pltpu.SemaphoreType.DMA,
            pltpu.SemaphoreType.DMA,
            pltpu.SemaphoreType.DMA,
            pltpu.SemaphoreType.DMA,
        ],
    )
    table_packed = pack_run(table)
    run = pl.kernel(
        _cbow_kernel,
        out_type=jax.ShapeDtypeStruct((B, D), jnp.float32),
        mesh=plsc.VectorSubcoreMesh(core_axis_name="c", subcore_axis_name="s"),
        compiler_params=pltpu.CompilerParams(needs_layout_passes=False,
                                             use_tc_tiling_on_sc=False),
        scratch_types=[
            pltpu.VMEM((B_PER_W * L,), jnp.int32),
            pltpu.VMEM((B_PER_W,), jnp.int32),
            pltpu.VMEM((B_PER_W,), jnp.float32),
            pltpu.VMEM((L, DW), jnp.int32),
            pltpu.VMEM((L, DW), jnp.int32),
            pltpu.VMEM((L, DW), jnp.int32),
            pltpu.VMEM((L, DW), jnp.int32),
            pltpu.VMEM((B_PER_W, D), jnp.float32),
            pltpu.SemaphoreType.DMA,
            pltpu.SemaphoreType.DMA,
            pltpu.SemaphoreType.DMA,
            pltpu.SemaphoreType.DMA,
        ],
    )
    return run(words, bs, table_packed)


def kernel(word_inputs_data, batch_sizes, embedding_table):
    words = word_inputs_data.astype(jnp.int32).reshape(B * L)
    bs = batch_sizes.astype(jnp.int32)
    return _cbow(words, bs, embedding_table.astype(jnp.float32))


# speedup vs baseline: 1.3027x; 1.3027x over previous
import functools

import jax
import jax.numpy as jnp
from jax import lax
from jax.experimental import pallas as pl
from jax.experimental.pallas import tpu as pltpu
from jax.experimental.pallas import tpu_sc as plsc

NC = 2
NS = 16
LANES = 16
NW = NC * NS

B = 4096
L = 200
D = 128
DW = D // 2
PARTS = (96, 104)
B_PER_W = B // NW
D_VECS = DW // LANES

HI_MASK = -65536

VOCAB = 100000
ROWS_W = VOCAB // NW
CH = 125
N_CH = ROWS_W // CH


def _rne_bf16(t):
    sixteen = jnp.uint32(16)
    return t + jnp.uint32(32767) + (lax.shift_right_logical(t, sixteen)
                                    & jnp.uint32(1))


def _pack_kernel(table_hbm, packed_hbm,
                 in_a, in_b, out_a, out_b,
                 semi_a, semi_b, semo_a, semo_b):
    wid = lax.axis_index("s") * NC + lax.axis_index("c")
    base = wid * ROWS_W

    def start_in(c, buf, sem):
        pltpu.make_async_copy(
            table_hbm.at[pl.ds(base + c * CH, CH)], buf, sem).start()

    def wait_in(buf, sem):
        pltpu.make_async_copy(
            table_hbm.at[pl.ds(base, CH)], buf, sem).wait()

    def start_out(c, buf, sem):
        pltpu.make_async_copy(
            buf, packed_hbm.at[pl.ds(base + c * CH, CH)], sem).start()

    def wait_out(buf, sem):
        pltpu.make_async_copy(
            buf, packed_hbm.at[pl.ds(base, CH)], sem).wait()

    def compute(src, dst):
        @functools.partial(plsc.parallel_loop, 0, CH, unroll=4)
        def _body(r):
            for c in range(D_VECS):
                t0 = plsc.bitcast(src[r, pl.ds(c * LANES, LANES)], jnp.uint32)
                t1 = plsc.bitcast(src[r, pl.ds(DW + c * LANES, LANES)],
                                  jnp.uint32)
                rlo = lax.shift_right_logical(t0, jnp.uint32(16))
                rhi = t1 & jnp.uint32(4294901760)
                dst[r, pl.ds(c * LANES, LANES)] = plsc.bitcast(rlo | rhi,
                                                               jnp.int32)

    start_in(0, in_a, semi_a)

    def pair_body(i, _):
        c0 = 2 * i
        c1 = 2 * i + 1

        @pl.when(c1 < N_CH)
        def _():
            start_in(c1, in_b, semi_b)
        wait_in(in_a, semi_a)

        @pl.when(c0 >= 2)
        def _():
            wait_out(out_a, semo_a)
        compute(in_a, out_a)
        start_out(c0, out_a, semo_a)

        @pl.when(c0 + 2 < N_CH)
        def _():
            start_in(c0 + 2, in_a, semi_a)

        @pl.when(c1 < N_CH)
        def _():
            wait_in(in_b, semi_b)

            @pl.when(c1 >= 2)
            def _():
                wait_out(out_b, semo_b)
            compute(in_b, out_b)
            start_out(c1, out_b, semo_b)
        return 0

    lax.fori_loop(0, (N_CH + 1) // 2, pair_body, 0)
    wait_out(out_a, semo_a)
    wait_out(out_b, semo_b)


def _cbow_kernel(words_hbm, bs_hbm, table_hbm, out_hbm,
                 idx_v, bs_v, recip_v, rows_a, rows_b, rows_c, rows_d, out_v,
                 sem_a, sem_b, sem_c, sem_d):
    wid = lax.axis_index("s") * NC + lax.axis_index("c")
    base = wid * B_PER_W

    pltpu.sync_copy(words_hbm.at[pl.ds(base * L, B_PER_W * L)], idx_v)
    pltpu.sync_copy(bs_hbm.at[pl.ds(base, B_PER_W)], bs_v)

    for c in range(B_PER_W // LANES):
        bsf = bs_v[pl.ds(c * LANES, LANES)].astype(jnp.float32)
        recip_v[pl.ds(c * LANES, LANES)] = 1.0 / bsf

    def start_gather(s, rows_ref, sem):
        off = 0
        for n in PARTS:
            pltpu.make_async_copy(
                table_hbm.at[idx_v.at[pl.ds(s * L + off, n)]],
                rows_ref.at[pl.ds(off, n)],
                sem,
            ).start()
            off += n

    def wait_gather(rows_ref, sem):
        off = 0
        for n in PARTS:
            pltpu.make_async_copy(
                table_hbm.at[idx_v.at[pl.ds(off, n)]],
                rows_ref.at[pl.ds(off, n)],
                sem,
            ).wait()
            off += n

    def accum_and_store(s, rows_ref):
        def body(p, accs):
            new = list(accs)
            for c in range(D_VECS):
                wa = plsc.bitcast(rows_ref[2 * p, pl.ds(c * LANES, LANES)],
                                  jnp.bfloat16)
                wb = plsc.bitcast(rows_ref[2 * p + 1, pl.ds(c * LANES, LANES)],
                                  jnp.bfloat16)
                t = plsc.bitcast(wa + wb, jnp.int32)
                lo = plsc.bitcast(lax.shift_left(t, 16), jnp.float32)
                hi = plsc.bitcast(t & HI_MASK, jnp.float32)
                new[2 * c] = new[2 * c] + lo
                new[2 * c + 1] = new[2 * c + 1] + hi
            return tuple(new)

        accs = lax.fori_loop(
            0, L // 2, body,
            tuple(jnp.zeros((LANES,), jnp.float32) for _ in range(2 * D_VECS)),
            unroll=2)
        r = plsc.load_gather(recip_v, [jnp.full((LANES,), s, dtype=jnp.int32)])
        for c in range(D_VECS):
            out_v[s, pl.ds(c * LANES, LANES)] = accs[2 * c] * r
            out_v[s, pl.ds(DW + c * LANES, LANES)] = accs[2 * c + 1] * r

    start_gather(0, rows_a, sem_a)
    start_gather(1, rows_b, sem_b)
    start_gather(2, rows_c, sem_c)

    bufs = ((rows_a, sem_a), (rows_b, sem_b), (rows_c, sem_c),
            (rows_d, sem_d))

    def quad_body(i, _):
        s = 4 * i
        start_gather(s + 3, rows_d, sem_d)
        for k, (buf, sem) in enumerate(bufs):
            wait_gather(buf, sem)
            accum_and_store(s + k, buf)

            if k < 3:
                @pl.when(s + k + 4 < B_PER_W)
                def _():
                    start_gather(s + k + 4, buf, sem)
        return 0

    lax.fori_loop(0, B_PER_W // 4, quad_body, 0)

    pltpu.sync_copy(out_v, out_hbm.at[pl.ds(base, B_PER_W)])


@jax.jit
def _cbow(words, bs, table):
    pack_run = pl.kernel(
        _pack_kernel,
        out_type=jax.ShapeDtypeStruct((VOCAB, DW), jnp.int32),
        mesh=plsc.VectorSubcoreMesh(core_axis_name="c", subcore_axis_name="s"),
        compiler_params=pltpu.CompilerParams(needs_layout_passes=False,
                                             use_tc_tiling_on_sc=False),
        scratch_types=[
            pltpu.VMEM((CH, D), jnp.float32),
            pltpu.VMEM((CH, D), jnp.float32),
            pltpu.VMEM((CH, DW), jnp.int32),
            pltpu.VMEM((CH, DW), jnp.int32),
            pltpu.SemaphoreType.DMA,
            pltpu.SemaphoreType.DMA,
            pltpu.SemaphoreType.DMA,
            pltpu.SemaphoreType.DMA,
        ],
    )
    table_packed = pack_run(table)
    run = pl.kernel(
        _cbow_kernel,
        out_type=jax.ShapeDtypeStruct((B, D), jnp.float32),
        mesh=plsc.VectorSubcoreMesh(core_axis_name="c", subcore_axis_name="s"),
        compiler_params=pltpu.CompilerParams(needs_layout_passes=False,
                                             use_tc_tiling_on_sc=False),
        scratch_types=[
            pltpu.VMEM((B_PER_W * L,), jnp.int32),
            pltpu.VMEM((B_PER_W,), jnp.int32),
            pltpu.VMEM((B_PER_W,), jnp.float32),
            pltpu.VMEM((L, DW), jnp.int32),
            pltpu.VMEM((L, DW), jnp.int32),
            pltpu.VMEM((L, DW), jnp.int32),
            pltpu.VMEM((L, DW), jnp.int32),
            pltpu.VMEM((B_PER_W, D), jnp.float32),
            pltpu.SemaphoreType.DMA,
            pltpu.SemaphoreType.DMA,
            pltpu.SemaphoreType.DMA,
            pltpu.SemaphoreType.DMA,
        ],
    )
    return run(words, bs, table_packed)


def kernel(word_inputs_data, batch_sizes, embedding_table):
    words = word_inputs_data.astype(jnp.int32).reshape(B * L)
    bs = batch_sizes.astype(jnp.int32)
    return _cbow(words, bs, embedding_table.astype(jnp.float32))
